# baseline (speedup 1.0000x reference)
import functools

import jax
import jax.numpy as jnp
from jax import lax
from jax.experimental import pallas as pl
from jax.experimental.pallas import tpu as pltpu
from jax.experimental.pallas import tpu_sc as plsc

VOCAB = 100000
HIDDEN = 768
NUM_TABLES = 3
B = 4 * 2048
NC, NS = 2, 16
NW = NC * NS
B_PER_W = B // NW
CHUNK = 32
NCHUNK = B_PER_W // CHUNK
DEPTH = 4
GLEAD = 2


@jax.jit
def _gather3(idx_flat, W0, W1, W2):
    out = jax.ShapeDtypeStruct((B, HIDDEN), jnp.float32)
    mesh = plsc.VectorSubcoreMesh(core_axis_name="c", subcore_axis_name="s")

    @functools.partial(
        pl.kernel,
        out_type=(out, out, out),
        mesh=mesh,
        scratch_types=[
            *[pltpu.VMEM((CHUNK,), jnp.int32) for _ in range(NCHUNK)],
            *[pltpu.VMEM((CHUNK, HIDDEN), jnp.float32) for _ in range(DEPTH)],
            *[pltpu.SemaphoreType.DMA for _ in range(2 * DEPTH)],
        ],
    )
    def k(idx_hbm, w0_hbm, w1_hbm, w2_hbm, o0_hbm, o1_hbm, o2_hbm, *scratch):
        idxs = scratch[:NCHUNK]
        bufs = scratch[NCHUNK:NCHUNK + DEPTH]
        gsems = scratch[NCHUNK + DEPTH:NCHUNK + 2 * DEPTH]
        wsems = scratch[NCHUNK + 2 * DEPTH:]
        tables = (w0_hbm, w1_hbm, w2_hbm)
        outs = (o0_hbm, o1_hbm, o2_hbm)

        wid = lax.axis_index("s") * NC + lax.axis_index("c")
        base = wid * B_PER_W

        for c in range(NCHUNK):
            pltpu.sync_copy(idx_hbm.at[pl.ds(base + c * CHUNK, CHUNK)], idxs[c])

        items = [(t, c) for t in range(NUM_TABLES) for c in range(NCHUNK)]
        n = len(items)
        gcopies, wcopies = {}, {}

        def gstart(m):
            t, c = items[m]
            gcopies[m] = pltpu.async_copy(
                tables[t].at[idxs[c]], bufs[m % DEPTH], gsems[m % DEPTH])

        def wstart(m):
            t, c = items[m]
            wcopies[m] = pltpu.async_copy(
                bufs[m % DEPTH],
                outs[t].at[pl.ds(base + c * CHUNK, CHUNK)],
                wsems[m % DEPTH])

        for m in range(min(GLEAD, n)):
            gstart(m)
        for j in range(n):
            gcopies[j].wait()
            wstart(j)
            m = j + GLEAD
            if m < n:
                if m >= DEPTH:
                    wcopies[m - DEPTH].wait()
                gstart(m)
        for m in range(max(0, n - DEPTH), n):
            wcopies[m].wait()

    return k(idx_flat, W0, W1, W2)


def kernel(inputs, W0, W1, W2):
    idx = inputs.reshape(-1).astype(jnp.int32)
    e0, e1, e2 = _gather3(idx, W0, W1, W2)
    shp = (*inputs.shape, HIDDEN)
    e0 = e0.reshape(shp)
    e1 = e1.reshape(shp)
    e2 = e2.reshape(shp)
    return (e0, e1, e2, e2, e1, e0)


# speedup vs baseline: 1.2032x; 1.2032x over previous
import functools

import jax
import jax.numpy as jnp
from jax import lax
from jax.experimental import pallas as pl
from jax.experimental.pallas import tpu as pltpu
from jax.experimental.pallas import tpu_sc as plsc

VOCAB = 100000
HIDDEN = 768
NUM_TABLES = 3
B = 4 * 2048
NC, NS = 2, 16
NW = NC * NS
B_PER_W = B // NW
CHUNK = 64
NCHUNK = B_PER_W // CHUNK
DEPTH = 2
GLEAD = 1


@jax.jit
def _gather3(idx_flat, W0, W1, W2):
    out = jax.ShapeDtypeStruct((B, HIDDEN), jnp.float32)
    mesh = plsc.VectorSubcoreMesh(core_axis_name="c", subcore_axis_name="s")

    @functools.partial(
        pl.kernel,
        out_type=(out,) * 6,
        mesh=mesh,
        scratch_types=[
            *[pltpu.VMEM((CHUNK,), jnp.int32) for _ in range(NCHUNK)],
            *[pltpu.VMEM((CHUNK, HIDDEN), jnp.float32) for _ in range(DEPTH)],
            *[pltpu.SemaphoreType.DMA for _ in range(2 * DEPTH)],
        ],
    )
    def k(idx_hbm, w0_hbm, w1_hbm, w2_hbm,
          o0_hbm, o1_hbm, o2_hbm, o3_hbm, o4_hbm, o5_hbm, *scratch):
        idxs = scratch[:NCHUNK]
        bufs = scratch[NCHUNK:NCHUNK + DEPTH]
        gsems = scratch[NCHUNK + DEPTH:NCHUNK + 2 * DEPTH]
        wsems = scratch[NCHUNK + 2 * DEPTH:]
        tables = (w0_hbm, w1_hbm, w2_hbm)
        outs = (o0_hbm, o1_hbm, o2_hbm, o3_hbm, o4_hbm, o5_hbm)

        wid = lax.axis_index("s") * NC + lax.axis_index("c")
        base = wid * B_PER_W

        for c in range(NCHUNK):
            pltpu.sync_copy(idx_hbm.at[pl.ds(base + c * CHUNK, CHUNK)], idxs[c])

        items = [(t, c) for t in range(NUM_TABLES) for c in range(NCHUNK)]
        n = len(items)
        gcopies, wcopies = {}, {}

        def gstart(m):
            t, c = items[m]
            gcopies[m] = pltpu.async_copy(
                tables[t].at[idxs[c]], bufs[m % DEPTH], gsems[m % DEPTH])

        def wstart(m):
            t, c = items[m]
            sl = pl.ds(base + c * CHUNK, CHUNK)
            wcopies[m] = (
                pltpu.async_copy(bufs[m % DEPTH], outs[t].at[sl],
                                 wsems[m % DEPTH]),
                pltpu.async_copy(bufs[m % DEPTH], outs[5 - t].at[sl],
                                 wsems[m % DEPTH]),
            )

        def wwait(m):
            wcopies[m][0].wait()
            wcopies[m][1].wait()

        for m in range(min(GLEAD, n)):
            gstart(m)
        for j in range(n):
            gcopies[j].wait()
            wstart(j)
            m = j + GLEAD
            if m < n:
                if m >= DEPTH:
                    wwait(m - DEPTH)
                gstart(m)
        for m in range(max(0, n - DEPTH), n):
            wwait(m)

    return k(idx_flat, W0, W1, W2)


def kernel(inputs, W0, W1, W2):
    idx = inputs.reshape(-1).astype(jnp.int32)
    outs = _gather3(idx, W0, W1, W2)
    shp = (*inputs.shape, HIDDEN)
    return tuple(o.reshape(shp) for o in outs)
